# baseline (speedup 1.0000x reference)
import functools

import jax
import jax.numpy as jnp
from jax import lax
from jax.experimental import pallas as pl
from jax.experimental.pallas import tpu as pltpu
from jax.experimental.pallas import tpu_sc as plsc

NC, NS, L = 2, 16, 16
NW = NC * NS

B, S, D = 4, 8192, 1024
SEQ_PER_W = S // NW
CH = 8
N_CH = SEQ_PER_W // CH
GRP = D // L
ITERS = [(c, b) for c in range(N_CH) for b in range(B)]


NBUF = 6


def _sc_add(x_hbm, pos_hbm, out_hbm, pos_v, x_v, *sems):
    wid = lax.axis_index("s") * NC + lax.axis_index("c")
    row0 = wid * SEQ_PER_W
    in_s = sems[:NBUF]
    out_s = sems[NBUF:2 * NBUF]
    pos_s = sems[2 * NBUF]

    def pos_in(c):
        return pltpu.async_copy(
            pos_hbm.at[pl.ds(row0 + c * CH, CH)], pos_v.at[c % 2], pos_s)

    def x_in(i):
        c, b = ITERS[i]
        return pltpu.async_copy(
            x_hbm.at[b, pl.ds(row0 + c * CH, CH)], x_v.at[i % NBUF],
            in_s[i % NBUF])

    def x_out(i):
        c, b = ITERS[i]
        return pltpu.async_copy(
            x_v.at[i % NBUF], out_hbm.at[b, pl.ds(row0 + c * CH, CH)],
            out_s[i % NBUF])

    def add_chunk(pbuf, xbuf):
        @plsc.parallel_loop(0, CH * GRP, unroll=8)
        def _(j):
            r = jnp.right_shift(j, 6)
            col = pl.multiple_of(
                jnp.left_shift(jnp.bitwise_and(j, GRP - 1), 4), L)
            v = pos_v[pbuf, r, pl.ds(col, L)]
            plsc.addupdate(x_v.at[xbuf, r, pl.ds(col, L)], v)

    n = len(ITERS)
    pend_pos = pos_in(0)
    pend_in = {}
    pend_out = {}
    for i in range(NBUF - 1):
        pend_in[i] = x_in(i)
    for i, (c, b) in enumerate(ITERS):
        if b == 0:
            pend_pos.wait()
        nxt = i + NBUF - 1
        if nxt < n:
            if nxt - NBUF >= 0:
                pend_out[nxt - NBUF].wait()
            pend_in[nxt] = x_in(nxt)
        pend_in[i].wait()
        if b == 0 and c + 1 < N_CH:
            pend_pos = pos_in(c + 1)
        add_chunk(c % 2, i % NBUF)
        pend_out[i] = x_out(i)
    for i in range(max(0, n - NBUF), n):
        if i in pend_out:
            pend_out[i].wait()


_sc_call = functools.partial(
    pl.kernel,
    out_type=jax.ShapeDtypeStruct((B, S, D), jnp.float32),
    mesh=plsc.VectorSubcoreMesh(core_axis_name="c", subcore_axis_name="s"),
    scratch_types=[
        pltpu.VMEM((2, CH, D), jnp.float32),
        pltpu.VMEM((NBUF, CH, D), jnp.float32),
    ] + [pltpu.SemaphoreType.DMA] * (2 * NBUF + 1),
)(_sc_add)


def kernel(x, pos_table):
    return _sc_call(x, pos_table)


# speedup vs baseline: 1.0294x; 1.0294x over previous
import functools

import jax
import jax.numpy as jnp
from jax import lax
from jax.experimental import pallas as pl
from jax.experimental.pallas import tpu as pltpu
from jax.experimental.pallas import tpu_sc as plsc

NC, NS, L = 2, 16, 16
NW = NC * NS

B, S, D = 4, 8192, 1024
SEQ_PER_W = S // NW
CH = 16
H = CH // 2
N_CH = SEQ_PER_W // CH
GRP = D // L
ITERS = [(c, b) for c in range(N_CH) for b in range(B)]


NBUF = 3


def _sc_add(x_hbm, pos_hbm, out_hbm, pos_v, x_v, *sems):
    wid = lax.axis_index("s") * NC + lax.axis_index("c")
    row0 = wid * SEQ_PER_W
    in_s = sems[:2 * NBUF]
    out_s = sems[2 * NBUF:3 * NBUF]
    pos_s = sems[3 * NBUF]

    def pos_in(c):
        return pltpu.async_copy(
            pos_hbm.at[pl.ds(row0 + c * CH, CH)], pos_v.at[c % 2], pos_s)

    def x_in(i, h):
        c, b = ITERS[i]
        return pltpu.async_copy(
            x_hbm.at[b, pl.ds(row0 + c * CH + h * H, H)],
            x_v.at[i % NBUF, pl.ds(h * H, H)],
            in_s[2 * (i % NBUF) + h])

    def x_out(i):
        c, b = ITERS[i]
        return pltpu.async_copy(
            x_v.at[i % NBUF], out_hbm.at[b, pl.ds(row0 + c * CH, CH)],
            out_s[i % NBUF])

    def add_half(pbuf, xbuf, h):
        @plsc.parallel_loop(0, H * GRP, unroll=8)
        def _(j):
            r = jnp.right_shift(j, 6) + h * H
            col = pl.multiple_of(
                jnp.left_shift(jnp.bitwise_and(j, GRP - 1), 4), L)
            v = pos_v[pbuf, r, pl.ds(col, L)]
            plsc.addupdate(x_v.at[xbuf, r, pl.ds(col, L)], v)

    n = len(ITERS)
    pend_pos = pos_in(0)
    pend_in = {}
    pend_out = {}
    for i in range(NBUF - 1):
        pend_in[i, 0] = x_in(i, 0)
        pend_in[i, 1] = x_in(i, 1)
    for i, (c, b) in enumerate(ITERS):
        if b == 0:
            pend_pos.wait()
        nxt = i + NBUF - 1
        if nxt < n:
            if nxt - NBUF >= 0:
                pend_out[nxt - NBUF].wait()
            pend_in[nxt, 0] = x_in(nxt, 0)
            pend_in[nxt, 1] = x_in(nxt, 1)
        if b == 0 and c + 1 < N_CH:
            pend_pos = pos_in(c + 1)
        for h in (0, 1):
            pend_in[i, h].wait()
            add_half(c % 2, i % NBUF, h)
        pend_out[i] = x_out(i)
    for i in range(max(0, n - NBUF), n):
        if i in pend_out:
            pend_out[i].wait()


_sc_call = functools.partial(
    pl.kernel,
    out_type=jax.ShapeDtypeStruct((B, S, D), jnp.float32),
    mesh=plsc.VectorSubcoreMesh(core_axis_name="c", subcore_axis_name="s"),
    scratch_types=[
        pltpu.VMEM((2, CH, D), jnp.float32),
        pltpu.VMEM((NBUF, CH, D), jnp.float32),
    ] + [pltpu.SemaphoreType.DMA] * (3 * NBUF + 1),
)(_sc_add)


def kernel(x, pos_table):
    return _sc_call(x, pos_table)


# speedup vs baseline: 1.1403x; 1.1077x over previous
import functools

import jax
import jax.numpy as jnp
from jax import lax
from jax.experimental import pallas as pl
from jax.experimental.pallas import tpu as pltpu
from jax.experimental.pallas import tpu_sc as plsc

NC, NS, L = 2, 16, 16
NW = NC * NS

B, S, D = 4, 8192, 1024
SEQ_PER_W = S // NW
CH = 16
H = CH // 2
N_CH = SEQ_PER_W // CH
GRP = D // L
ITERS = [(c, b) for c in range(N_CH) for b in range(B)]


NBUF = 3


def _sc_add(x_hbm, pos_hbm, out_hbm, pos_v, x_v, *sems):
    wid = lax.axis_index("s") * NC + lax.axis_index("c")
    row0 = wid * SEQ_PER_W
    in_s = sems[:NBUF]
    out_s = sems[NBUF:3 * NBUF]
    pos_s = sems[3 * NBUF]

    def pos_in(c):
        return pltpu.async_copy(
            pos_hbm.at[pl.ds(row0 + c * CH, CH)], pos_v.at[c % 2], pos_s)

    def x_in(i):
        c, b = ITERS[i]
        return pltpu.async_copy(
            x_hbm.at[b, pl.ds(row0 + c * CH, CH)], x_v.at[i % NBUF],
            in_s[i % NBUF])

    def x_out(i, h):
        c, b = ITERS[i]
        return pltpu.async_copy(
            x_v.at[i % NBUF, pl.ds(h * H, H)],
            out_hbm.at[b, pl.ds(row0 + c * CH + h * H, H)],
            out_s[2 * (i % NBUF) + h])

    def add_half(pbuf, xbuf, h):
        @plsc.parallel_loop(0, H * GRP, unroll=8)
        def _(j):
            r = jnp.right_shift(j, 6) + h * H
            col = pl.multiple_of(
                jnp.left_shift(jnp.bitwise_and(j, GRP - 1), 4), L)
            v = pos_v[pbuf, r, pl.ds(col, L)]
            plsc.addupdate(x_v.at[xbuf, r, pl.ds(col, L)], v)

    n = len(ITERS)
    pend_pos = pos_in(0)
    pend_in = {}
    pend_out = {}
    for i in range(NBUF - 1):
        pend_in[i] = x_in(i)
    for i, (c, b) in enumerate(ITERS):
        if b == 0:
            pend_pos.wait()
        nxt = i + NBUF - 1
        if nxt < n:
            if nxt - NBUF >= 0:
                pend_out[nxt - NBUF, 0].wait()
                pend_out[nxt - NBUF, 1].wait()
            pend_in[nxt] = x_in(nxt)
        pend_in[i].wait()
        if b == 0 and c + 1 < N_CH:
            pend_pos = pos_in(c + 1)
        add_half(c % 2, i % NBUF, 0)
        pend_out[i, 0] = x_out(i, 0)
        add_half(c % 2, i % NBUF, 1)
        pend_out[i, 1] = x_out(i, 1)
    for i in range(max(0, n - NBUF), n):
        for h in (0, 1):
            if (i, h) in pend_out:
                pend_out[i, h].wait()


_sc_call = functools.partial(
    pl.kernel,
    out_type=jax.ShapeDtypeStruct((B, S, D), jnp.float32),
    mesh=plsc.VectorSubcoreMesh(core_axis_name="c", subcore_axis_name="s"),
    scratch_types=[
        pltpu.VMEM((2, CH, D), jnp.float32),
        pltpu.VMEM((NBUF, CH, D), jnp.float32),
    ] + [pltpu.SemaphoreType.DMA] * (3 * NBUF + 1),
)(_sc_add)


def kernel(x, pos_table):
    return _sc_call(x, pos_table)
